# Initial kernel scaffold; baseline (speedup 1.0000x reference)
import jax
import jax.numpy as jnp
from jax.experimental import pallas as pl


def kernel(edge_index, n_feats, batch, params):
    raise NotImplementedError("write your pallas kernel here")


# speedup vs baseline: 16.6506x; 16.6506x over previous
import functools

import jax
import jax.numpy as jnp
from jax import lax
from jax.experimental import pallas as pl
from jax.experimental.pallas import tpu as pltpu
from jax.experimental.pallas import tpu_sc as plsc

N = 50000
E = 800000
IN_DIM = 38
HID = 64
D = 64
G = 64
NC = 2
NS = 16
L = 16
HALF = N // NC
TRASH = HALF
ROWS_PT = 1568
ROWS_LAST = HALF - 15 * ROWS_PT
EPT = E // NS
C = 128
NFULL = EPT // C
TAIL = EPT - NFULL * C


def _edge_body(t_hbm, u_hbm, v_hbm, z_hbm, out_hbm,
               uidx, vraw, vloc, rows, uidx2, vloc2, rows2, acc, sem):
    c = lax.axis_index("c")
    s = lax.axis_index("s")
    node_base = c * HALF
    row0 = s * ROWS_PT

    @pl.when(s < NS - 1)
    def _():
        pltpu.sync_copy(z_hbm.at[pl.ds(0, ROWS_PT)], acc.at[pl.ds(row0, ROWS_PT)])

    @pl.when(s == NS - 1)
    def _():
        pltpu.sync_copy(z_hbm.at[pl.ds(0, ROWS_LAST)],
                        acc.at[pl.ds(row0, ROWS_LAST)])

    plsc.subcore_barrier()

    ebase = s * EPT

    def chunk(i, _):
        off = ebase + i * C
        pltpu.sync_copy(u_hbm.at[pl.ds(off, C)], uidx)
        pltpu.sync_copy(v_hbm.at[pl.ds(off, C)], vraw)
        cp = pltpu.async_copy(t_hbm.at[uidx], rows, sem)
        for j in range(C // L):
            w = vraw[pl.ds(j * L, L)]
            wl = w - node_base
            ok = (wl >= 0) & (wl < HALF)
            vloc[pl.ds(j * L, L)] = jnp.where(ok, wl, TRASH)
        cp.wait()
        pltpu.sync_copy(rows, acc.at[vloc], add=True)
        return 0

    lax.fori_loop(0, NFULL, chunk, 0)

    off = ebase + NFULL * C
    pltpu.sync_copy(u_hbm.at[pl.ds(off, TAIL)], uidx2)
    pltpu.sync_copy(v_hbm.at[pl.ds(off, TAIL)], vraw.at[pl.ds(0, TAIL)])
    cp = pltpu.async_copy(t_hbm.at[uidx2], rows2, sem)
    for j in range(TAIL // L):
        w = vraw[pl.ds(j * L, L)]
        wl = w - node_base
        ok = (wl >= 0) & (wl < HALF)
        vloc2[pl.ds(j * L, L)] = jnp.where(ok, wl, TRASH)
    cp.wait()
    pltpu.sync_copy(rows2, acc.at[vloc2], add=True)

    plsc.subcore_barrier()

    gbase = node_base + row0

    @pl.when(s < NS - 1)
    def _():
        pltpu.sync_copy(acc.at[pl.ds(row0, ROWS_PT)],
                        out_hbm.at[pl.ds(gbase, ROWS_PT)])

    @pl.when(s == NS - 1)
    def _():
        pltpu.sync_copy(acc.at[pl.ds(row0, ROWS_LAST)],
                        out_hbm.at[pl.ds(gbase, ROWS_LAST)])


_edge_kernel = functools.partial(
    pl.kernel,
    out_type=jax.ShapeDtypeStruct((N, D), jnp.float32),
    mesh=plsc.VectorSubcoreMesh(core_axis_name="c", subcore_axis_name="s",
                                num_cores=NC, num_subcores=NS),
    compiler_params=pltpu.CompilerParams(use_tc_tiling_on_sc=False),
    scratch_types=[
        pltpu.VMEM((C,), jnp.int32),
        pltpu.VMEM((C,), jnp.int32),
        pltpu.VMEM((C,), jnp.int32),
        pltpu.VMEM((C, D), jnp.float32),
        pltpu.VMEM((TAIL,), jnp.int32),
        pltpu.VMEM((TAIL,), jnp.int32),
        pltpu.VMEM((TAIL, D), jnp.float32),
        pltpu.VMEM_SHARED((HALF + 1, D), jnp.float32),
        pltpu.SemaphoreType.DMA,
    ],
)(_edge_body)


BLK = 2000
NB = N // BLK

_f32 = jnp.float32
_blk = lambda d: pl.BlockSpec((BLK, d), lambda i: (i, 0))
_full = lambda r, d: pl.BlockSpec((r, d), lambda i: (0, 0))


def _leaky_ln(m, w, b, g, bb):
    h = jnp.dot(m, w, preferred_element_type=_f32) + b
    mu = jnp.mean(h, axis=1, keepdims=True)
    var = jnp.mean((h - mu) * (h - mu), axis=1, keepdims=True)
    h = (h - mu) * lax.rsqrt(var + 1e-5) * g + bb
    return jnp.where(h >= 0, h, 0.01 * h)


def _enc_mm_body(nf_ref, w_ref, b_ref, xp_ref, ssum_ref, ssq_ref):
    i = pl.program_id(0)
    x = jnp.dot(nf_ref[...], w_ref[...], preferred_element_type=_f32) + b_ref[...]
    xp_ref[...] = x

    @pl.when(i == 0)
    def _():
        ssum_ref[...] = jnp.zeros_like(ssum_ref)
        ssq_ref[...] = jnp.zeros_like(ssq_ref)

    ssum_ref[...] += jnp.sum(x, axis=0, keepdims=True)
    ssq_ref[...] += jnp.sum(x * x, axis=0, keepdims=True)


_enc_mm = pl.pallas_call(
    _enc_mm_body,
    grid=(NB,),
    in_specs=[_blk(IN_DIM), _full(IN_DIM, HID), _full(1, HID)],
    out_specs=(_blk(HID), _full(1, HID), _full(1, HID)),
    out_shape=(jax.ShapeDtypeStruct((N, HID), _f32),
               jax.ShapeDtypeStruct((1, HID), _f32),
               jax.ShapeDtypeStruct((1, HID), _f32)),
)


def _bn_alpha_body(xp_ref, ssum_ref, ssq_ref, g_ref, bb_ref, au_ref,
                   x_ref, al_ref, am_ref):
    i = pl.program_id(0)
    mu = ssum_ref[...] * (1.0 / N)
    var = ssq_ref[...] * (1.0 / N) - mu * mu
    x = (xp_ref[...] - mu) * lax.rsqrt(var + 1e-5) * g_ref[...] + bb_ref[...]
    x_ref[...] = x
    al = jnp.sum(x * au_ref[...], axis=1, keepdims=True)
    al_ref[...] = al

    @pl.when(i == 0)
    def _():
        am_ref[...] = jnp.full_like(am_ref, -jnp.inf)

    am_ref[...] = jnp.maximum(am_ref[...],
                              jnp.max(al, axis=(0, 1), keepdims=True))


_bn_alpha = pl.pallas_call(
    _bn_alpha_body,
    grid=(NB,),
    in_specs=[_blk(HID), _full(1, HID), _full(1, HID), _full(1, HID),
              _full(1, HID), _full(1, HID)],
    out_specs=(_blk(HID), _blk(1), _full(1, 1)),
    out_shape=(jax.ShapeDtypeStruct((N, HID), _f32),
               jax.ShapeDtypeStruct((N, 1), _f32),
               jax.ShapeDtypeStruct((1, 1), _f32)),
)


def _t_body(x_ref, al_ref, am_ref, t_ref):
    t_ref[...] = x_ref[...] * jnp.exp(al_ref[...] - am_ref[...])


_t_kernel = pl.pallas_call(
    _t_body,
    grid=(NB,),
    in_specs=[_blk(HID), _blk(1), _full(1, 1)],
    out_specs=_blk(HID),
    out_shape=jax.ShapeDtypeStruct((N, D), _f32),
)


def _layer_body(m_ref, w_ref, b_ref, g_ref, bb_ref, au_ref, fs_ref,
                x_ref, fso_ref, al_ref, am_ref):
    i = pl.program_id(0)
    x = _leaky_ln(m_ref[...], w_ref[...], b_ref[...], g_ref[...], bb_ref[...])
    x_ref[...] = x
    fso_ref[...] = fs_ref[...] + x
    al = jnp.sum(x * au_ref[...], axis=1, keepdims=True)
    al_ref[...] = al

    @pl.when(i == 0)
    def _():
        am_ref[...] = jnp.full_like(am_ref, -jnp.inf)

    am_ref[...] = jnp.maximum(am_ref[...],
                              jnp.max(al, axis=(0, 1), keepdims=True))


_layer_kernel = pl.pallas_call(
    _layer_body,
    grid=(NB,),
    in_specs=[_blk(D), _full(HID, HID), _full(1, HID), _full(1, HID),
              _full(1, HID), _full(1, HID), _blk(HID)],
    out_specs=(_blk(HID), _blk(HID), _blk(1), _full(1, 1)),
    out_shape=(jax.ShapeDtypeStruct((N, HID), _f32),
               jax.ShapeDtypeStruct((N, HID), _f32),
               jax.ShapeDtypeStruct((N, 1), _f32),
               jax.ShapeDtypeStruct((1, 1), _f32)),
)


def _final_body(m_ref, w_ref, b_ref, g_ref, bb_ref, fs_ref, bt_ref,
                g2_ref, b2_ref, sums_ref, cnt_ref, out_ref):
    i = pl.program_id(0)
    x4 = _leaky_ln(m_ref[...], w_ref[...], b_ref[...], g_ref[...], bb_ref[...])
    n_out = (fs_ref[...] + x4) * 0.2
    gi = lax.broadcasted_iota(jnp.int32, (G, BLK), 0)
    oh = (gi == bt_ref[0]).astype(_f32)

    @pl.when(i == 0)
    def _():
        sums_ref[...] = jnp.zeros_like(sums_ref)
        cnt_ref[...] = jnp.zeros_like(cnt_ref)

    sums_ref[...] += jnp.dot(oh, n_out, preferred_element_type=_f32)
    cnt_ref[...] += jnp.sum(oh, axis=1, keepdims=True)

    @pl.when(i == NB - 1)
    def _():
        prot = sums_ref[...] / jnp.maximum(cnt_ref[...], 1.0)
        mu = jnp.mean(prot, axis=0, keepdims=True)
        var = jnp.mean((prot - mu) * (prot - mu), axis=0, keepdims=True)
        prot = (prot - mu) * lax.rsqrt(var + 1e-5) * g2_ref[...] + b2_ref[...]
        out_ref[...] = jnp.where(prot >= 0, prot, 0.01 * prot)


_final_kernel = pl.pallas_call(
    _final_body,
    grid=(NB,),
    in_specs=[_blk(D), _full(HID, HID), _full(1, HID), _full(1, HID),
              _full(1, HID), _blk(HID), pl.BlockSpec((1, 1, BLK), lambda i: (i, 0, 0)),
              _full(1, HID), _full(1, HID)],
    out_specs=(_full(G, HID), _full(G, 1), _full(G, HID)),
    out_shape=(jax.ShapeDtypeStruct((G, HID), _f32),
               jax.ShapeDtypeStruct((G, 1), _f32),
               jax.ShapeDtypeStruct((G, HID), _f32)),
)


def kernel(edge_index, n_feats, batch, params):
    u = edge_index[0]
    v = edge_index[1]
    zeros = jnp.zeros((ROWS_PT, D), jnp.float32)
    row = lambda a: a.reshape(1, -1)

    aU = [params['layers'][l]['a_W'][:HID, 0].reshape(1, HID)
          for l in range(4)]

    xp, ssum, ssq = _enc_mm(n_feats, params['enc_W'], row(params['enc_b']))
    x, al, am = _bn_alpha(xp, ssum, ssq, row(params['bn1_g']),
                          row(params['bn1_b']), aU[0])
    fsum = x
    T = _t_kernel(x, al, am)

    for l in range(3):
        p = params['layers'][l]
        M = _edge_kernel(T, u, v, zeros)
        x, fsum, al, am = _layer_kernel(M, p['lin_W'], row(p['lin_b']),
                                        row(p['ln_g']), row(p['ln_b']),
                                        aU[l + 1], fsum)
        T = _t_kernel(x, al, am)

    p = params['layers'][3]
    M = _edge_kernel(T, u, v, zeros)
    _, _, out = _final_kernel(M, p['lin_W'], row(p['lin_b']),
                              row(p['ln_g']), row(p['ln_b']),
                              fsum, batch.reshape(NB, 1, BLK),
                              row(params['bn2_g']), row(params['bn2_b']))
    return out


# speedup vs baseline: 25.7435x; 1.5461x over previous
import functools

import jax
import jax.numpy as jnp
from jax import lax
from jax.experimental import pallas as pl
from jax.experimental.pallas import tpu as pltpu
from jax.experimental.pallas import tpu_sc as plsc

N = 50000
E = 800000
IN_DIM = 38
HID = 64
D = 64
G = 64
NC = 2
NS = 16
L = 16
HALF = N // NC
TRASH = HALF
ROWS_PT = 1568
ROWS_LAST = HALF - 15 * ROWS_PT
EPT = E // NS
C = 128
NFULL = EPT // C
TAIL = EPT - NFULL * C


LOOPK = NFULL // 3 - 1


def _edge_body(t_hbm, u_hbm, v_hbm, z_hbm, out_hbm,
               ub0, ub1, ub2, vb0, vb1, vb2, rw0, rw1, rw2, vloc,
               uidx2, vloc2, rows2, acc,
               si0, si1, si2, sg0, sg1, sg2, sem):
    c = lax.axis_index("c")
    s = lax.axis_index("s")
    node_base = c * HALF
    row0 = s * ROWS_PT

    U = [ub0, ub1, ub2]
    V = [vb0, vb1, vb2]
    RW = [rw0, rw1, rw2]
    SI = [si0, si1, si2]
    SG = [sg0, sg1, sg2]

    @pl.when(s < NS - 1)
    def _():
        pltpu.sync_copy(z_hbm.at[pl.ds(0, ROWS_PT)], acc.at[pl.ds(row0, ROWS_PT)])

    @pl.when(s == NS - 1)
    def _():
        pltpu.sync_copy(z_hbm.at[pl.ds(0, ROWS_LAST)],
                        acc.at[pl.ds(row0, ROWS_LAST)])

    plsc.subcore_barrier()

    ebase = s * EPT

    def issue_idx(ch, b):
        off = ebase + ch * C
        pltpu.make_async_copy(u_hbm.at[pl.ds(off, C)], U[b], SI[b]).start()
        pltpu.make_async_copy(v_hbm.at[pl.ds(off, C)], V[b], SI[b]).start()

    def wait_idx(b):
        pltpu.make_async_copy(u_hbm.at[pl.ds(0, C)], U[b], SI[b]).wait()
        pltpu.make_async_copy(v_hbm.at[pl.ds(0, C)], V[b], SI[b]).wait()

    def issue_gather(b):
        pltpu.make_async_copy(t_hbm.at[U[b]], RW[b], SG[b]).start()

    def wait_gather(b):
        pltpu.make_async_copy(t_hbm.at[U[b]], RW[b], SG[b]).wait()

    def vloc_comp(b):
        for j in range(C // L):
            w = V[b][pl.ds(j * L, L)]
            wl = w - node_base
            ok = (wl >= 0) & (wl < HALF)
            vloc[pl.ds(j * L, L)] = jnp.where(ok, wl, TRASH)

    def scatter(b):
        pltpu.sync_copy(RW[b], acc.at[vloc], add=True)

    issue_idx(0, 0)
    issue_idx(1, 1)
    issue_idx(2, 2)
    wait_idx(0)
    issue_gather(0)

    def body3(k, _):
        c0 = 3 * k
        wait_idx(1)
        issue_gather(1)
        vloc_comp(0)
        wait_gather(0)
        scatter(0)
        issue_idx(c0 + 3, 0)
        wait_idx(2)
        issue_gather(2)
        vloc_comp(1)
        wait_gather(1)
        scatter(1)
        issue_idx(c0 + 4, 1)
        wait_idx(0)
        issue_gather(0)
        vloc_comp(2)
        wait_gather(2)
        scatter(2)
        issue_idx(c0 + 5, 2)
        return 0

    lax.fori_loop(0, LOOPK, body3, 0)

    wait_idx(1)
    issue_gather(1)
    vloc_comp(0)
    wait_gather(0)
    scatter(0)
    wait_idx(2)
    issue_gather(2)
    vloc_comp(1)
    wait_gather(1)
    scatter(1)
    vloc_comp(2)
    wait_gather(2)
    scatter(2)

    off = ebase + NFULL * C
    pltpu.sync_copy(u_hbm.at[pl.ds(off, TAIL)], uidx2)
    pltpu.sync_copy(v_hbm.at[pl.ds(off, TAIL)], vb0.at[pl.ds(0, TAIL)])
    cp = pltpu.async_copy(t_hbm.at[uidx2], rows2, sem)
    for j in range(TAIL // L):
        w = vb0[pl.ds(j * L, L)]
        wl = w - node_base
        ok = (wl >= 0) & (wl < HALF)
        vloc2[pl.ds(j * L, L)] = jnp.where(ok, wl, TRASH)
    cp.wait()
    pltpu.sync_copy(rows2, acc.at[vloc2], add=True)

    plsc.subcore_barrier()

    gbase = node_base + row0

    @pl.when(s < NS - 1)
    def _():
        pltpu.sync_copy(acc.at[pl.ds(row0, ROWS_PT)],
                        out_hbm.at[pl.ds(gbase, ROWS_PT)])

    @pl.when(s == NS - 1)
    def _():
        pltpu.sync_copy(acc.at[pl.ds(row0, ROWS_LAST)],
                        out_hbm.at[pl.ds(gbase, ROWS_LAST)])


_edge_kernel = functools.partial(
    pl.kernel,
    out_type=jax.ShapeDtypeStruct((N, D), jnp.float32),
    mesh=plsc.VectorSubcoreMesh(core_axis_name="c", subcore_axis_name="s",
                                num_cores=NC, num_subcores=NS),
    compiler_params=pltpu.CompilerParams(use_tc_tiling_on_sc=False),
    scratch_types=[
        pltpu.VMEM((C,), jnp.int32),
        pltpu.VMEM((C,), jnp.int32),
        pltpu.VMEM((C,), jnp.int32),
        pltpu.VMEM((C,), jnp.int32),
        pltpu.VMEM((C,), jnp.int32),
        pltpu.VMEM((C,), jnp.int32),
        pltpu.VMEM((C, D), jnp.float32),
        pltpu.VMEM((C, D), jnp.float32),
        pltpu.VMEM((C, D), jnp.float32),
        pltpu.VMEM((C,), jnp.int32),
        pltpu.VMEM((TAIL,), jnp.int32),
        pltpu.VMEM((TAIL,), jnp.int32),
        pltpu.VMEM((TAIL, D), jnp.float32),
        pltpu.VMEM_SHARED((HALF + 1, D), jnp.float32),
        pltpu.SemaphoreType.DMA,
        pltpu.SemaphoreType.DMA,
        pltpu.SemaphoreType.DMA,
        pltpu.SemaphoreType.DMA,
        pltpu.SemaphoreType.DMA,
        pltpu.SemaphoreType.DMA,
        pltpu.SemaphoreType.DMA,
    ],
)(_edge_body)


BLK = 2000
NB = N // BLK

_f32 = jnp.float32
_blk = lambda d: pl.BlockSpec((BLK, d), lambda i: (i, 0))
_full = lambda r, d: pl.BlockSpec((r, d), lambda i: (0, 0))


def _leaky_ln(m, w, b, g, bb):
    h = jnp.dot(m, w, preferred_element_type=_f32) + b
    mu = jnp.mean(h, axis=1, keepdims=True)
    var = jnp.mean((h - mu) * (h - mu), axis=1, keepdims=True)
    h = (h - mu) * lax.rsqrt(var + 1e-5) * g + bb
    return jnp.where(h >= 0, h, 0.01 * h)


def _enc_mm_body(nf_ref, w_ref, b_ref, xp_ref, ssum_ref, ssq_ref):
    i = pl.program_id(0)
    x = jnp.dot(nf_ref[...], w_ref[...], preferred_element_type=_f32) + b_ref[...]
    xp_ref[...] = x

    @pl.when(i == 0)
    def _():
        ssum_ref[...] = jnp.zeros_like(ssum_ref)
        ssq_ref[...] = jnp.zeros_like(ssq_ref)

    ssum_ref[...] += jnp.sum(x, axis=0, keepdims=True)
    ssq_ref[...] += jnp.sum(x * x, axis=0, keepdims=True)


_enc_mm = pl.pallas_call(
    _enc_mm_body,
    grid=(NB,),
    in_specs=[_blk(IN_DIM), _full(IN_DIM, HID), _full(1, HID)],
    out_specs=(_blk(HID), _full(1, HID), _full(1, HID)),
    out_shape=(jax.ShapeDtypeStruct((N, HID), _f32),
               jax.ShapeDtypeStruct((1, HID), _f32),
               jax.ShapeDtypeStruct((1, HID), _f32)),
)


def _bn_alpha_body(xp_ref, ssum_ref, ssq_ref, g_ref, bb_ref, au_ref,
                   x_ref, al_ref, am_ref):
    i = pl.program_id(0)
    mu = ssum_ref[...] * (1.0 / N)
    var = ssq_ref[...] * (1.0 / N) - mu * mu
    x = (xp_ref[...] - mu) * lax.rsqrt(var + 1e-5) * g_ref[...] + bb_ref[...]
    x_ref[...] = x
    al = jnp.sum(x * au_ref[...], axis=1, keepdims=True)
    al_ref[...] = al

    @pl.when(i == 0)
    def _():
        am_ref[...] = jnp.full_like(am_ref, -jnp.inf)

    am_ref[...] = jnp.maximum(am_ref[...],
                              jnp.max(al, axis=(0, 1), keepdims=True))


_bn_alpha = pl.pallas_call(
    _bn_alpha_body,
    grid=(NB,),
    in_specs=[_blk(HID), _full(1, HID), _full(1, HID), _full(1, HID),
              _full(1, HID), _full(1, HID)],
    out_specs=(_blk(HID), _blk(1), _full(1, 1)),
    out_shape=(jax.ShapeDtypeStruct((N, HID), _f32),
               jax.ShapeDtypeStruct((N, 1), _f32),
               jax.ShapeDtypeStruct((1, 1), _f32)),
)


def _t_body(x_ref, al_ref, am_ref, t_ref):
    t_ref[...] = x_ref[...] * jnp.exp(al_ref[...] - am_ref[...])


_t_kernel = pl.pallas_call(
    _t_body,
    grid=(NB,),
    in_specs=[_blk(HID), _blk(1), _full(1, 1)],
    out_specs=_blk(HID),
    out_shape=jax.ShapeDtypeStruct((N, D), _f32),
)


def _layer_body(m_ref, w_ref, b_ref, g_ref, bb_ref, au_ref, fs_ref,
                x_ref, fso_ref, al_ref, am_ref):
    i = pl.program_id(0)
    x = _leaky_ln(m_ref[...], w_ref[...], b_ref[...], g_ref[...], bb_ref[...])
    x_ref[...] = x
    fso_ref[...] = fs_ref[...] + x
    al = jnp.sum(x * au_ref[...], axis=1, keepdims=True)
    al_ref[...] = al

    @pl.when(i == 0)
    def _():
        am_ref[...] = jnp.full_like(am_ref, -jnp.inf)

    am_ref[...] = jnp.maximum(am_ref[...],
                              jnp.max(al, axis=(0, 1), keepdims=True))


_layer_kernel = pl.pallas_call(
    _layer_body,
    grid=(NB,),
    in_specs=[_blk(D), _full(HID, HID), _full(1, HID), _full(1, HID),
              _full(1, HID), _full(1, HID), _blk(HID)],
    out_specs=(_blk(HID), _blk(HID), _blk(1), _full(1, 1)),
    out_shape=(jax.ShapeDtypeStruct((N, HID), _f32),
               jax.ShapeDtypeStruct((N, HID), _f32),
               jax.ShapeDtypeStruct((N, 1), _f32),
               jax.ShapeDtypeStruct((1, 1), _f32)),
)


def _final_body(m_ref, w_ref, b_ref, g_ref, bb_ref, fs_ref, bt_ref,
                g2_ref, b2_ref, sums_ref, cnt_ref, out_ref):
    i = pl.program_id(0)
    x4 = _leaky_ln(m_ref[...], w_ref[...], b_ref[...], g_ref[...], bb_ref[...])
    n_out = (fs_ref[...] + x4) * 0.2
    gi = lax.broadcasted_iota(jnp.int32, (G, BLK), 0)
    oh = (gi == bt_ref[0]).astype(_f32)

    @pl.when(i == 0)
    def _():
        sums_ref[...] = jnp.zeros_like(sums_ref)
        cnt_ref[...] = jnp.zeros_like(cnt_ref)

    sums_ref[...] += jnp.dot(oh, n_out, preferred_element_type=_f32)
    cnt_ref[...] += jnp.sum(oh, axis=1, keepdims=True)

    @pl.when(i == NB - 1)
    def _():
        prot = sums_ref[...] / jnp.maximum(cnt_ref[...], 1.0)
        mu = jnp.mean(prot, axis=0, keepdims=True)
        var = jnp.mean((prot - mu) * (prot - mu), axis=0, keepdims=True)
        prot = (prot - mu) * lax.rsqrt(var + 1e-5) * g2_ref[...] + b2_ref[...]
        out_ref[...] = jnp.where(prot >= 0, prot, 0.01 * prot)


_final_kernel = pl.pallas_call(
    _final_body,
    grid=(NB,),
    in_specs=[_blk(D), _full(HID, HID), _full(1, HID), _full(1, HID),
              _full(1, HID), _blk(HID), pl.BlockSpec((1, 1, BLK), lambda i: (i, 0, 0)),
              _full(1, HID), _full(1, HID)],
    out_specs=(_full(G, HID), _full(G, 1), _full(G, HID)),
    out_shape=(jax.ShapeDtypeStruct((G, HID), _f32),
               jax.ShapeDtypeStruct((G, 1), _f32),
               jax.ShapeDtypeStruct((G, HID), _f32)),
)


def kernel(edge_index, n_feats, batch, params):
    u = edge_index[0]
    v = edge_index[1]
    zeros = jnp.zeros((ROWS_PT, D), jnp.float32)
    row = lambda a: a.reshape(1, -1)

    aU = [params['layers'][l]['a_W'][:HID, 0].reshape(1, HID)
          for l in range(4)]

    xp, ssum, ssq = _enc_mm(n_feats, params['enc_W'], row(params['enc_b']))
    x, al, am = _bn_alpha(xp, ssum, ssq, row(params['bn1_g']),
                          row(params['bn1_b']), aU[0])
    fsum = x
    T = _t_kernel(x, al, am)

    for l in range(3):
        p = params['layers'][l]
        M = _edge_kernel(T, u, v, zeros)
        x, fsum, al, am = _layer_kernel(M, p['lin_W'], row(p['lin_b']),
                                        row(p['ln_g']), row(p['ln_b']),
                                        aU[l + 1], fsum)
        T = _t_kernel(x, al, am)

    p = params['layers'][3]
    M = _edge_kernel(T, u, v, zeros)
    _, _, out = _final_kernel(M, p['lin_W'], row(p['lin_b']),
                              row(p['ln_g']), row(p['ln_b']),
                              fsum, batch.reshape(NB, 1, BLK),
                              row(params['bn2_g']), row(params['bn2_b']))
    return out
